# baseline (speedup 1.0000x reference)
import functools

import jax
import jax.numpy as jnp
from jax import lax
from jax.experimental import pallas as pl
from jax.experimental.pallas import tpu as pltpu
from jax.experimental.pallas import tpu_sc as plsc

H = 128
B = 256
N = 100000
OUT = 10

SC_ROWS = 48000
TC_ROWS = N - SC_ROWS

NW = 32
RB = 128
SC_NBLK = SC_ROWS // RB
BLK_PER_W = SC_NBLK // NW
EXTRA = SC_NBLK - NW * BLK_PER_W
MAXBLK = BLK_PER_W + 1

SROWS = B + 16

R = 4000
TC_NBLK = TC_ROWS // R
TC_OFF = SC_ROWS // R


def _sigmoid(x):
    return 1.0 / (1.0 + jnp.exp(-x))


def _tanh(x):
    return 1.0 - 2.0 / (1.0 + jnp.exp(2.0 * x))


def _sc_pool_kernel(bih_hbm, bhh_hbm, hid_hbm, seg_hbm, s_out,
                    bih_v, bhh_v, seg_buf, hid_buf, s_arr, sh_s,
                    sem_h0, sem_h1, sem_s0, sem_s1):
    wid = lax.axis_index("c") * 16 + lax.axis_index("s")
    nblk = jnp.where(wid < EXTRA, BLK_PER_W + 1, BLK_PER_W)
    base_blk = wid * BLK_PER_W + jnp.minimum(wid, EXTRA)

    iota = lax.iota(jnp.int32, 16)
    zeros = iota.astype(jnp.float32) * 0.0
    zeros_i = iota * 0
    perms = [jnp.bitwise_xor(iota, sh) for sh in (1, 2, 4, 8)]
    masks = [(iota & sh) == 0 for sh in (1, 2, 4, 8)]

    def _lane_sums(vecs):
        for perm, mask in zip(perms, masks):
            nxt = []
            for i in range(len(vecs) // 2):
                x, y = vecs[2 * i], vecs[2 * i + 1]
                nxt.append(jnp.where(mask, x + x[perm], y + y[perm]))
            vecs = nxt
        return vecs[0]

    def _zero(i, c):
        for k in range(8):
            s_arr[i, pl.ds(k * 16, 16)] = zeros
        return c
    lax.fori_loop(0, SROWS, _zero, 0)

    pltpu.sync_copy(bih_hbm, bih_v)
    pltpu.sync_copy(bhh_hbm, bhh_v)
    q = []
    for j in range(8):
        ig = bih_v[pl.ds(j * 16, 16)] + bhh_v[pl.ds(j * 16, 16)]
        gg = bih_v[pl.ds(2 * H + j * 16, 16)] + bhh_v[pl.ds(2 * H + j * 16, 16)]
        og = bih_v[pl.ds(3 * H + j * 16, 16)] + bhh_v[pl.ds(3 * H + j * 16, 16)]
        c = _sigmoid(ig) * _tanh(gg)
        q.append(_sigmoid(og) * _tanh(c))

    def _issue(blk, slot, sem_h, sem_s):
        st = blk * RB
        pltpu.make_async_copy(hid_hbm.at[pl.ds(st, RB), :],
                              hid_buf.at[slot], sem_h).start()
        pltpu.make_async_copy(seg_hbm.at[pl.ds(st, RB)],
                              seg_buf.at[slot], sem_s).start()

    def _wait(blk, slot, sem_h, sem_s):
        st = blk * RB
        pltpu.make_async_copy(hid_hbm.at[pl.ds(st, RB), :],
                              hid_buf.at[slot], sem_h).wait()
        pltpu.make_async_copy(seg_hbm.at[pl.ds(st, RB)],
                              seg_buf.at[slot], sem_s).wait()

    def _process(blk, slot):
        def _grp(g, c):
            base = g * 16
            seg16 = seg_buf[slot, pl.ds(base, 16)]
            accs = []
            for m in range(16):
                r = base + m
                rows = [hid_buf[slot, r, pl.ds(j * 16, 16)] for j in range(8)]
                p0 = rows[0] * q[0] + rows[1] * q[1]
                p1 = rows[2] * q[2] + rows[3] * q[3]
                p2 = rows[4] * q[4] + rows[5] * q[5]
                p3 = rows[6] * q[6] + rows[7] * q[7]
                accs.append((p0 + p1) + (p2 + p3))
            w16 = jnp.exp(_lane_sums(accs))
            for m in range(16):
                r = base + m
                rows = [hid_buf[slot, r, pl.ds(j * 16, 16)] for j in range(8)]
                wsp = w16[zeros_i + m]
                seg_splat = seg16[zeros_i + m]
                for j in range(8):
                    plsc.addupdate_scatter(s_arr,
                                           [seg_splat, iota + (j * 16)],
                                           wsp * rows[j])
                plsc.addupdate_scatter(
                    s_arr, [lax.shift_right_logical(seg_splat, 7) + B,
                            jnp.bitwise_and(seg_splat, 127)],
                    wsp, mask=iota == 0)
            return c
        lax.fori_loop(0, RB // 16, _grp, 0)

    _issue(base_blk, 0, sem_h0, sem_s0)

    @pl.when(nblk > 1)
    def _():
        _issue(base_blk + 1, 1, sem_h1, sem_s1)

    def _pair(j, c):
        b0 = base_blk + 2 * j
        b1 = b0 + 1
        b2 = b0 + 2
        b3 = b0 + 3

        @pl.when(2 * j < nblk)
        def _():
            _wait(b0, 0, sem_h0, sem_s0)
            _process(b0, 0)

            @pl.when(b2 - base_blk < nblk)
            def _():
                _issue(b2, 0, sem_h0, sem_s0)

        @pl.when(2 * j + 1 < nblk)
        def _():
            _wait(b1, 1, sem_h1, sem_s1)
            _process(b1, 1)

            @pl.when(b3 - base_blk < nblk)
            def _():
                _issue(b3, 1, sem_h1, sem_s1)

        return c

    lax.fori_loop(0, (MAXBLK + 1) // 2, _pair, 0)

    sid = lax.axis_index("s")
    cid = lax.axis_index("c")

    @pl.when(sid == 0)
    def _seed():
        pltpu.sync_copy(s_arr, sh_s)

    plsc.subcore_barrier()

    @pl.when(sid != 0)
    def _accum():
        cps = [pltpu.async_copy(s_arr.at[pl.ds(c2 * 16, 16)],
                                sh_s.at[iota + (c2 * 16)], sem_h0, add=True)
               for c2 in range(SROWS // 16)]
        for cp in cps:
            cp.wait()

    plsc.subcore_barrier()
    pltpu.sync_copy(sh_s.at[pl.ds(sid * 16, 16)],
                    s_out.at[cid, pl.ds(sid * 16, 16)])

    @pl.when(sid == 0)
    def _tail():
        pltpu.sync_copy(sh_s.at[pl.ds(B, SROWS - B)],
                        s_out.at[cid, pl.ds(B, SROWS - B)])


@functools.partial(pl.kernel,
                   out_type=jax.ShapeDtypeStruct((2, SROWS, H), jnp.float32),
                   mesh=plsc.VectorSubcoreMesh(core_axis_name="c",
                                               subcore_axis_name="s"),
                   compiler_params=pltpu.CompilerParams(
                       needs_layout_passes=False),
                   scratch_types=[
                       pltpu.VMEM((4 * H,), jnp.float32),
                       pltpu.VMEM((4 * H,), jnp.float32),
                       pltpu.VMEM((2, RB), jnp.int32),
                       pltpu.VMEM((2, RB, H), jnp.float32),
                       pltpu.VMEM((SROWS, H), jnp.float32),
                       pltpu.VMEM_SHARED((SROWS, H), jnp.float32),
                       pltpu.SemaphoreType.DMA,
                       pltpu.SemaphoreType.DMA,
                       pltpu.SemaphoreType.DMA,
                       pltpu.SemaphoreType.DMA,
                   ])
def _sc_pool(bih, bhh, hid, seg, s_out, *scratch):
    _sc_pool_kernel(bih, bhh, hid, seg, s_out, *scratch)


def _tc_pool_kernel(hid_ref, seg_ref, bih_ref, bhh_ref, d_ref, s_ref,
                    d_acc, s_acc):
    i = pl.program_id(0)

    @pl.when(i == 0)
    def _init():
        d_acc[...] = jnp.zeros_like(d_acc)
        s_acc[...] = jnp.zeros_like(s_acc)

    gates = bih_ref[...] + bhh_ref[...]
    i_g = jax.nn.sigmoid(gates[0:1, :])
    g_g = jnp.tanh(gates[2:3, :])
    o_g = jax.nn.sigmoid(gates[3:4, :])
    q = o_g * jnp.tanh(i_g * g_g)

    hid_blk = hid_ref[...]
    seg = seg_ref[...].reshape(1, R)

    e = lax.dot_general(q, hid_blk, (((1,), (1,)), ((), ())),
                        preferred_element_type=jnp.float32)
    w = jnp.exp(e)

    seg_iota = lax.broadcasted_iota(jnp.int32, (B, 1), 0)
    w_mat = jnp.where(seg == seg_iota, w, 0.0)

    d_acc[...] += jnp.sum(w_mat, axis=1, keepdims=True)
    s_acc[...] += lax.dot_general(w_mat, hid_blk, (((1,), (0,)), ((), ())),
                                  preferred_element_type=jnp.float32)

    @pl.when(i == TC_NBLK - 1)
    def _out():
        d_ref[...] = d_acc[...]
        s_ref[...] = s_acc[...]


def _tc_pool(hid, seg3d, bih2, bhh2):
    return pl.pallas_call(
        _tc_pool_kernel,
        grid=(TC_NBLK,),
        in_specs=[
            pl.BlockSpec((R, H), lambda i: (i + TC_OFF, 0)),
            pl.BlockSpec((1, 1, R), lambda i: (i + TC_OFF, 0, 0)),
            pl.BlockSpec((4, H), lambda i: (0, 0)),
            pl.BlockSpec((4, H), lambda i: (0, 0)),
        ],
        out_specs=[
            pl.BlockSpec((B, 1), lambda i: (0, 0)),
            pl.BlockSpec((B, H), lambda i: (0, 0)),
        ],
        out_shape=[
            jax.ShapeDtypeStruct((B, 1), jnp.float32),
            jax.ShapeDtypeStruct((B, H), jnp.float32),
        ],
        scratch_shapes=[
            pltpu.VMEM((B, 1), jnp.float32),
            pltpu.VMEM((B, H), jnp.float32),
        ],
    )(hid, seg3d, bih2, bhh2)


def _merge_kernel(dp_ref, sp_ref, dt_ref, st_ref, bih_ref, bhh_ref,
                  w1_ref, b1_ref, w2_ref, b2_ref, out_ref):
    gates = bih_ref[...] + bhh_ref[...]
    i_g = jax.nn.sigmoid(gates[0:1, :])
    g_g = jnp.tanh(gates[2:3, :])
    o_g = jax.nn.sigmoid(gates[3:4, :])
    q = o_g * jnp.tanh(i_g * g_g)

    s_sum = jnp.sum(sp_ref[...].reshape(2, B, H), axis=0) + st_ref[...]
    d_sum = (jnp.sum(dp_ref[...], axis=0).reshape(B, 1) + dt_ref[...])
    readout = jnp.where(d_sum > 0.0, s_sum / d_sum, 0.0)

    w1 = w1_ref[...]
    q_part = lax.dot_general(q, w1[:, :H], (((1,), (1,)), ((), ())),
                             preferred_element_type=jnp.float32)
    r_part = lax.dot_general(readout, w1[:, H:], (((1,), (1,)), ((), ())),
                             preferred_element_type=jnp.float32)
    pre1 = q_part + r_part + b1_ref[...]
    x1 = jnp.where(pre1 > 0.0, pre1, jnp.exp(pre1) - 1.0)
    x2 = lax.dot_general(x1, w2_ref[...], (((1,), (1,)), ((), ())),
                         preferred_element_type=jnp.float32)
    pre2 = x2 + b2_ref[...]
    out_ref[...] = jnp.where(pre2 > 0.0, pre2, jnp.exp(pre2) - 1.0)


def _merge(d_part, s_part, d_tc, s_tc, bih2, bhh2, W1, b1r, W2p, b2p):
    return pl.pallas_call(
        _merge_kernel,
        in_specs=[
            pl.BlockSpec((2, B), lambda: (0, 0)),
            pl.BlockSpec((2 * B, H), lambda: (0, 0)),
            pl.BlockSpec((B, 1), lambda: (0, 0)),
            pl.BlockSpec((B, H), lambda: (0, 0)),
            pl.BlockSpec((4, H), lambda: (0, 0)),
            pl.BlockSpec((4, H), lambda: (0, 0)),
            pl.BlockSpec((H, 2 * H), lambda: (0, 0)),
            pl.BlockSpec((1, H), lambda: (0, 0)),
            pl.BlockSpec((16, H), lambda: (0, 0)),
            pl.BlockSpec((1, 16), lambda: (0, 0)),
        ],
        out_specs=pl.BlockSpec((B, 16), lambda: (0, 0)),
        out_shape=jax.ShapeDtypeStruct((B, 16), jnp.float32),
    )(d_part, s_part, d_tc, s_tc, bih2, bhh2, W1, b1r, W2p, b2p)


@jax.jit
def _run(hid, seg, seg3d, bih, bhh, bih2, bhh2, W1, b1r, W2p, b2p):
    s_all = _sc_pool(bih, bhh, hid, seg)
    d_tc, s_tc = _tc_pool(hid, seg3d, bih2, bhh2)
    d_part = s_all[:, B:B + 2, :].reshape(2, B)
    s_part = s_all[:, :B, :].reshape(2 * B, H)
    out16 = _merge(d_part, s_part, d_tc, s_tc,
                   bih2, bhh2, W1, b1r, W2p, b2p)
    return out16[:, :OUT]


def kernel(hid, segment_ids, W_ih, W_hh, b_ih, b_hh, W1, b1, W2, b2):
    seg = segment_ids.astype(jnp.int32)
    seg3d = seg.reshape(N // R, 1, R)
    bih2 = b_ih.reshape(4, H)
    bhh2 = b_hh.reshape(4, H)
    b1r = b1.reshape(1, H)
    W2p = jnp.zeros((16, H), jnp.float32).at[:OUT].set(W2)
    b2p = jnp.zeros((1, 16), jnp.float32).at[0, :OUT].set(b2)
    return _run(hid, seg, seg3d, b_ih, b_hh, bih2, bhh2, W1, b1r, W2p, b2p)


# speedup vs baseline: 1.1431x; 1.1431x over previous
import functools

import jax
import jax.numpy as jnp
from jax import lax
from jax.experimental import pallas as pl
from jax.experimental.pallas import tpu as pltpu
from jax.experimental.pallas import tpu_sc as plsc

H = 128
B = 256
N = 100000
OUT = 10

SC_ROWS = 32000
TC_ROWS = N - SC_ROWS

NW = 32
RB = 128
SC_NBLK = SC_ROWS // RB
BLK_PER_W = SC_NBLK // NW
EXTRA = SC_NBLK - NW * BLK_PER_W
MAXBLK = BLK_PER_W + 1

SROWS = B + 16

R = 4000
TC_NBLK = TC_ROWS // R
TC_OFF = SC_ROWS // R


def _sigmoid(x):
    return 1.0 / (1.0 + jnp.exp(-x))


def _tanh(x):
    return 1.0 - 2.0 / (1.0 + jnp.exp(2.0 * x))


def _sc_pool_kernel(bih_hbm, bhh_hbm, hid_hbm, seg_hbm, s_out,
                    bih_v, bhh_v, seg_buf, hid_buf, s_arr, sh_s,
                    sem_h0, sem_h1, sem_s0, sem_s1):
    wid = lax.axis_index("c") * 16 + lax.axis_index("s")
    nblk = jnp.where(wid < EXTRA, BLK_PER_W + 1, BLK_PER_W)
    base_blk = wid * BLK_PER_W + jnp.minimum(wid, EXTRA)

    iota = lax.iota(jnp.int32, 16)
    zeros = iota.astype(jnp.float32) * 0.0
    zeros_i = iota * 0
    perms = [jnp.bitwise_xor(iota, sh) for sh in (1, 2, 4, 8)]
    masks = [(iota & sh) == 0 for sh in (1, 2, 4, 8)]

    def _lane_sums(vecs):
        for perm, mask in zip(perms, masks):
            nxt = []
            for i in range(len(vecs) // 2):
                x, y = vecs[2 * i], vecs[2 * i + 1]
                nxt.append(jnp.where(mask, x + x[perm], y + y[perm]))
            vecs = nxt
        return vecs[0]

    def _zero(i, c):
        for k in range(8):
            s_arr[i, pl.ds(k * 16, 16)] = zeros
        return c
    lax.fori_loop(0, SROWS, _zero, 0)

    pltpu.sync_copy(bih_hbm, bih_v)
    pltpu.sync_copy(bhh_hbm, bhh_v)
    q = []
    for j in range(8):
        ig = bih_v[pl.ds(j * 16, 16)] + bhh_v[pl.ds(j * 16, 16)]
        gg = bih_v[pl.ds(2 * H + j * 16, 16)] + bhh_v[pl.ds(2 * H + j * 16, 16)]
        og = bih_v[pl.ds(3 * H + j * 16, 16)] + bhh_v[pl.ds(3 * H + j * 16, 16)]
        c = _sigmoid(ig) * _tanh(gg)
        q.append(_sigmoid(og) * _tanh(c))

    def _issue(blk, slot, sem_h, sem_s):
        st = blk * RB
        pltpu.make_async_copy(hid_hbm.at[pl.ds(st, RB), :],
                              hid_buf.at[slot], sem_h).start()
        pltpu.make_async_copy(seg_hbm.at[pl.ds(st, RB)],
                              seg_buf.at[slot], sem_s).start()

    def _wait(blk, slot, sem_h, sem_s):
        st = blk * RB
        pltpu.make_async_copy(hid_hbm.at[pl.ds(st, RB), :],
                              hid_buf.at[slot], sem_h).wait()
        pltpu.make_async_copy(seg_hbm.at[pl.ds(st, RB)],
                              seg_buf.at[slot], sem_s).wait()

    def _process(blk, slot):
        def _grp(g, c):
            base = g * 16
            seg16 = seg_buf[slot, pl.ds(base, 16)]
            accs = []
            for m in range(16):
                r = base + m
                rows = [hid_buf[slot, r, pl.ds(j * 16, 16)] for j in range(8)]
                p0 = rows[0] * q[0] + rows[1] * q[1]
                p1 = rows[2] * q[2] + rows[3] * q[3]
                p2 = rows[4] * q[4] + rows[5] * q[5]
                p3 = rows[6] * q[6] + rows[7] * q[7]
                accs.append((p0 + p1) + (p2 + p3))
            w16 = jnp.exp(_lane_sums(accs))
            for m in range(16):
                r = base + m
                rows = [hid_buf[slot, r, pl.ds(j * 16, 16)] for j in range(8)]
                wsp = w16[zeros_i + m]
                seg_splat = seg16[zeros_i + m]
                for j in range(8):
                    plsc.addupdate_scatter(s_arr,
                                           [seg_splat, iota + (j * 16)],
                                           wsp * rows[j])
                plsc.addupdate_scatter(
                    s_arr, [lax.shift_right_logical(seg_splat, 7) + B,
                            jnp.bitwise_and(seg_splat, 127)],
                    wsp, mask=iota == 0)
            return c
        lax.fori_loop(0, RB // 16, _grp, 0)

    _issue(base_blk, 0, sem_h0, sem_s0)

    @pl.when(nblk > 1)
    def _():
        _issue(base_blk + 1, 1, sem_h1, sem_s1)

    def _pair(j, c):
        b0 = base_blk + 2 * j
        b1 = b0 + 1
        b2 = b0 + 2
        b3 = b0 + 3

        @pl.when(2 * j < nblk)
        def _():
            _wait(b0, 0, sem_h0, sem_s0)
            _process(b0, 0)

            @pl.when(b2 - base_blk < nblk)
            def _():
                _issue(b2, 0, sem_h0, sem_s0)

        @pl.when(2 * j + 1 < nblk)
        def _():
            _wait(b1, 1, sem_h1, sem_s1)
            _process(b1, 1)

            @pl.when(b3 - base_blk < nblk)
            def _():
                _issue(b3, 1, sem_h1, sem_s1)

        return c

    lax.fori_loop(0, (MAXBLK + 1) // 2, _pair, 0)

    sid = lax.axis_index("s")
    cid = lax.axis_index("c")

    @pl.when(sid == 0)
    def _seed():
        pltpu.sync_copy(s_arr, sh_s)

    plsc.subcore_barrier()

    @pl.when(sid != 0)
    def _accum():
        cps = [pltpu.async_copy(s_arr.at[pl.ds(c2 * 16, 16)],
                                sh_s.at[iota + (c2 * 16)], sem_h0, add=True)
               for c2 in range(SROWS // 16)]
        for cp in cps:
            cp.wait()

    plsc.subcore_barrier()
    pltpu.sync_copy(sh_s.at[pl.ds(sid * 16, 16)],
                    s_out.at[cid, pl.ds(sid * 16, 16)])

    @pl.when(sid == 0)
    def _tail():
        pltpu.sync_copy(sh_s.at[pl.ds(B, SROWS - B)],
                        s_out.at[cid, pl.ds(B, SROWS - B)])


@functools.partial(pl.kernel,
                   out_type=jax.ShapeDtypeStruct((2, SROWS, H), jnp.float32),
                   mesh=plsc.VectorSubcoreMesh(core_axis_name="c",
                                               subcore_axis_name="s"),
                   compiler_params=pltpu.CompilerParams(
                       needs_layout_passes=False),
                   scratch_types=[
                       pltpu.VMEM((4 * H,), jnp.float32),
                       pltpu.VMEM((4 * H,), jnp.float32),
                       pltpu.VMEM((2, RB), jnp.int32),
                       pltpu.VMEM((2, RB, H), jnp.float32),
                       pltpu.VMEM((SROWS, H), jnp.float32),
                       pltpu.VMEM_SHARED((SROWS, H), jnp.float32),
                       pltpu.SemaphoreType.DMA,
                       pltpu.SemaphoreType.DMA,
                       pltpu.SemaphoreType.DMA,
                       pltpu.SemaphoreType.DMA,
                   ])
def _sc_pool(bih, bhh, hid, seg, s_out, *scratch):
    _sc_pool_kernel(bih, bhh, hid, seg, s_out, *scratch)


def _tc_pool_kernel(hid_ref, seg_ref, bih_ref, bhh_ref, d_ref, s_ref,
                    d_acc, s_acc):
    i = pl.program_id(0)

    @pl.when(i == 0)
    def _init():
        d_acc[...] = jnp.zeros_like(d_acc)
        s_acc[...] = jnp.zeros_like(s_acc)

    gates = bih_ref[...] + bhh_ref[...]
    i_g = jax.nn.sigmoid(gates[0:1, :])
    g_g = jnp.tanh(gates[2:3, :])
    o_g = jax.nn.sigmoid(gates[3:4, :])
    q = o_g * jnp.tanh(i_g * g_g)

    hid_blk = hid_ref[...]
    seg = seg_ref[...].reshape(1, R)

    e = lax.dot_general(q, hid_blk, (((1,), (1,)), ((), ())),
                        preferred_element_type=jnp.float32)
    w = jnp.exp(e)

    seg_iota = lax.broadcasted_iota(jnp.int32, (B, 1), 0)
    w_mat = jnp.where(seg == seg_iota, w, 0.0)

    d_acc[...] += jnp.sum(w_mat, axis=1, keepdims=True)
    s_acc[...] += lax.dot_general(w_mat, hid_blk, (((1,), (0,)), ((), ())),
                                  preferred_element_type=jnp.float32)

    @pl.when(i == TC_NBLK - 1)
    def _out():
        d_ref[...] = d_acc[...]
        s_ref[...] = s_acc[...]


def _tc_pool(hid, seg3d, bih2, bhh2):
    return pl.pallas_call(
        _tc_pool_kernel,
        grid=(TC_NBLK,),
        in_specs=[
            pl.BlockSpec((R, H), lambda i: (i + TC_OFF, 0)),
            pl.BlockSpec((1, 1, R), lambda i: (i + TC_OFF, 0, 0)),
            pl.BlockSpec((4, H), lambda i: (0, 0)),
            pl.BlockSpec((4, H), lambda i: (0, 0)),
        ],
        out_specs=[
            pl.BlockSpec((B, 1), lambda i: (0, 0)),
            pl.BlockSpec((B, H), lambda i: (0, 0)),
        ],
        out_shape=[
            jax.ShapeDtypeStruct((B, 1), jnp.float32),
            jax.ShapeDtypeStruct((B, H), jnp.float32),
        ],
        scratch_shapes=[
            pltpu.VMEM((B, 1), jnp.float32),
            pltpu.VMEM((B, H), jnp.float32),
        ],
    )(hid, seg3d, bih2, bhh2)


def _merge_kernel(dp_ref, sp_ref, dt_ref, st_ref, bih_ref, bhh_ref,
                  w1_ref, b1_ref, w2_ref, b2_ref, out_ref):
    gates = bih_ref[...] + bhh_ref[...]
    i_g = jax.nn.sigmoid(gates[0:1, :])
    g_g = jnp.tanh(gates[2:3, :])
    o_g = jax.nn.sigmoid(gates[3:4, :])
    q = o_g * jnp.tanh(i_g * g_g)

    s_sum = jnp.sum(sp_ref[...].reshape(2, B, H), axis=0) + st_ref[...]
    d_sum = (jnp.sum(dp_ref[...], axis=0).reshape(B, 1) + dt_ref[...])
    readout = jnp.where(d_sum > 0.0, s_sum / d_sum, 0.0)

    w1 = w1_ref[...]
    q_part = lax.dot_general(q, w1[:, :H], (((1,), (1,)), ((), ())),
                             preferred_element_type=jnp.float32)
    r_part = lax.dot_general(readout, w1[:, H:], (((1,), (1,)), ((), ())),
                             preferred_element_type=jnp.float32)
    pre1 = q_part + r_part + b1_ref[...]
    x1 = jnp.where(pre1 > 0.0, pre1, jnp.exp(pre1) - 1.0)
    x2 = lax.dot_general(x1, w2_ref[...], (((1,), (1,)), ((), ())),
                         preferred_element_type=jnp.float32)
    pre2 = x2 + b2_ref[...]
    out_ref[...] = jnp.where(pre2 > 0.0, pre2, jnp.exp(pre2) - 1.0)


def _merge(d_part, s_part, d_tc, s_tc, bih2, bhh2, W1, b1r, W2p, b2p):
    return pl.pallas_call(
        _merge_kernel,
        in_specs=[
            pl.BlockSpec((2, B), lambda: (0, 0)),
            pl.BlockSpec((2 * B, H), lambda: (0, 0)),
            pl.BlockSpec((B, 1), lambda: (0, 0)),
            pl.BlockSpec((B, H), lambda: (0, 0)),
            pl.BlockSpec((4, H), lambda: (0, 0)),
            pl.BlockSpec((4, H), lambda: (0, 0)),
            pl.BlockSpec((H, 2 * H), lambda: (0, 0)),
            pl.BlockSpec((1, H), lambda: (0, 0)),
            pl.BlockSpec((16, H), lambda: (0, 0)),
            pl.BlockSpec((1, 16), lambda: (0, 0)),
        ],
        out_specs=pl.BlockSpec((B, 16), lambda: (0, 0)),
        out_shape=jax.ShapeDtypeStruct((B, 16), jnp.float32),
    )(d_part, s_part, d_tc, s_tc, bih2, bhh2, W1, b1r, W2p, b2p)


@jax.jit
def _run(hid, seg, seg3d, bih, bhh, bih2, bhh2, W1, b1r, W2p, b2p):
    s_all = _sc_pool(bih, bhh, hid, seg)
    d_tc, s_tc = _tc_pool(hid, seg3d, bih2, bhh2)
    d_part = s_all[:, B:B + 2, :].reshape(2, B)
    s_part = s_all[:, :B, :].reshape(2 * B, H)
    out16 = _merge(d_part, s_part, d_tc, s_tc,
                   bih2, bhh2, W1, b1r, W2p, b2p)
    return out16[:, :OUT]


def kernel(hid, segment_ids, W_ih, W_hh, b_ih, b_hh, W1, b1, W2, b2):
    seg = segment_ids.astype(jnp.int32)
    seg3d = seg.reshape(N // R, 1, R)
    bih2 = b_ih.reshape(4, H)
    bhh2 = b_hh.reshape(4, H)
    b1r = b1.reshape(1, H)
    W2p = jnp.zeros((16, H), jnp.float32).at[:OUT].set(W2)
    b2p = jnp.zeros((1, 16), jnp.float32).at[0, :OUT].set(b2)
    return _run(hid, seg, seg3d, b_ih, b_hh, bih2, bhh2, W1, b1r, W2p, b2p)
